# baseline (speedup 1.0000x reference)
import jax
import jax.numpy as jnp
from jax import lax
from jax.experimental import pallas as pl
from jax.experimental.pallas import tpu as pltpu

_BB = 4


def _conv_s2(a, w, ho):
    bb, hh, _, c = a.shape
    wo = ho
    ap = a.reshape(bb, hh, wo, 2, c)
    ae, ao = ap[:, :, :, 0, :], ap[:, :, :, 1, :]
    zc = jnp.zeros((bb, hh, 1, c), a.dtype)
    ao_l = jnp.concatenate([zc, ao], axis=2)
    ae_r = jnp.concatenate([ae, zc], axis=2)
    zr = jnp.zeros((bb, 1, wo + 1, c), a.dtype)

    def _rows(p):
        p = jnp.concatenate([zr, p, zr], axis=1)
        return p.reshape(bb, ho + 1, 2, wo + 1, c)

    ao_l, ae_r = _rows(ao_l), _rows(ae_r)
    acc = None
    for kh in range(4):
        for kw in range(4):
            plane = ao_l if kw % 2 == 0 else ae_r
            s0 = 0 if kw < 2 else 1
            t = plane[:, kh // 2:kh // 2 + ho, kh % 2, s0:s0 + wo, :]
            d = jnp.dot(t.reshape(bb * ho * wo, c),
                        w[(kh * 4 + kw) * c:(kh * 4 + kw + 1) * c, :],
                        preferred_element_type=jnp.float32)
            acc = d if acc is None else acc + d
    return acc


def _encoder_kernel(cols0_ref, w0_ref, w1_ref, w2_ref, w3_ref, w4_ref, w5_ref,
                    out_ref):
    f32 = jnp.float32
    a = jnp.dot(cols0_ref[...], w0_ref[...], preferred_element_type=f32)
    a = jnp.maximum(a, 0.0).reshape(_BB, 32, 32, 48)
    for w_ref, ho in ((w1_ref, 16), (w2_ref, 8), (w3_ref, 4)):
        a = _conv_s2(a, w_ref[...], ho)
        a = jnp.maximum(a, 0.0)
        a = a.reshape(_BB, ho, ho, w_ref.shape[1])
    cols = jnp.concatenate(
        [a[:, h, w, :] for h in range(4) for w in range(4)], axis=1)
    a = jnp.maximum(jnp.dot(cols, w4_ref[...], preferred_element_type=f32),
                    0.0)
    out_ref[...] = jnp.tanh(
        jnp.dot(a, w5_ref[...], preferred_element_type=f32))[None]


def kernel(x, w0, w1, w2, w3, w4, w5):
    b = x.shape[0]
    patches = lax.conv_general_dilated_patches(
        x, (4, 4), (2, 2), [(1, 1), (1, 1)])
    cols0 = patches.transpose(0, 2, 3, 1).reshape(b * 32 * 32, 48)
    wms = [w0.reshape(w0.shape[0], -1).T] +           [jnp.transpose(w, (2, 3, 1, 0)).reshape(-1, w.shape[0])
           for w in (w1, w2, w3, w4, w5)]
    out = pl.pallas_call(
        _encoder_kernel,
        out_shape=jax.ShapeDtypeStruct((b // _BB, _BB, 128), jnp.float32),
        grid=(b // _BB,),
        in_specs=[pl.BlockSpec((_BB * 1024, 48), lambda i: (i, 0))] +
                 [pl.BlockSpec(wm.shape, lambda i: (0, 0)) for wm in wms],
        out_specs=pl.BlockSpec((1, _BB, 128), lambda i: (i, 0, 0)),
        compiler_params=pltpu.CompilerParams(
            dimension_semantics=("parallel",),
            vmem_limit_bytes=100 * 1024 * 1024),
    )(cols0, *wms)
    return out.reshape(b, 128)
